# baseline (speedup 1.0000x reference)
import functools

import jax
import jax.numpy as jnp
from jax import lax
from jax.experimental import pallas as pl
from jax.experimental.pallas import tpu as pltpu
from jax.experimental.pallas import tpu_sc as plsc

N = 10000
NP = 10112
CNP = 10240
H = 128
E = 320000

NC = 2
NS = 16
NW = NC * NS
EP = 327680
EPT = EP // NW
C = 64
NCHUNK = EPT // C
NBUF = 4
RPS = NP // NS
CRPS = CNP // NS
ZB = 79


def _sc_agg_body(with_count, feat_hbm, src_hbm, dst_hbm, *refs):
    if with_count:
        (acc_out, cnt_out, ones_v, zc_v, zb_v, acc_sh, cnt_sh) = refs[:7]
        refs = refs[7:]
    else:
        (acc_out, zb_v, acc_sh) = refs[:3]
        refs = refs[3:]
    srcs = refs[0:NBUF]
    dsts = refs[NBUF:2 * NBUF]
    rows = refs[2 * NBUF:3 * NBUF]
    gsem = refs[3 * NBUF:4 * NBUF]
    ssem = refs[4 * NBUF]
    if with_count:
        csem = refs[4 * NBUF + 1]

    c = lax.axis_index("c")
    s = lax.axis_index("s")
    wid = c * NS + s

    zeros16 = jnp.zeros((16,), jnp.float32)

    def zrow(r, carry):
        for cc in range(H // 16):
            zb_v[r, pl.ds(cc * 16, 16)] = zeros16
        return carry
    lax.fori_loop(0, ZB, zrow, 0)

    if with_count:
        ones16 = jnp.ones((16,), jnp.float32)

        def orow(r, carry):
            ones_v[pl.ds(r * 16, 16)] = ones16
            return carry
        lax.fori_loop(0, C // 16, orow, 0)

        def zcrow(r, carry):
            zc_v[pl.ds(r * 16, 16)] = zeros16
            return carry
        lax.fori_loop(0, CRPS // 16, zcrow, 0)

    row0 = s * RPS
    for z in range(RPS // ZB):
        pltpu.sync_copy(zb_v, acc_sh.at[pl.ds(row0 + z * ZB, ZB)])
    crow0 = s * CRPS
    if with_count:
        pltpu.sync_copy(zc_v, cnt_sh.at[pl.ds(crow0, CRPS)])
    plsc.subcore_barrier()

    base = wid * EPT
    for b in range(NBUF):
        off = base + b * C
        pltpu.sync_copy(src_hbm.at[pl.ds(off, C)], srcs[b])
        pltpu.sync_copy(dst_hbm.at[pl.ds(off, C)], dsts[b])
        pltpu.async_copy(feat_hbm.at[srcs[b]], rows[b], gsem[b])

    def quad(g, carry):
        ib = g * NBUF
        for b in range(NBUF):
            i = ib + b
            pltpu.make_async_copy(feat_hbm.at[srcs[b]], rows[b],
                                  gsem[b]).wait()
            if with_count:
                pltpu.async_copy(ones_v, cnt_sh.at[dsts[b]], csem, add=True)
            pltpu.async_copy(rows[b], acc_sh.at[dsts[b]], ssem, add=True)

            @pl.when(i + NBUF < NCHUNK)
            def _():
                off = base + (i + NBUF) * C
                pltpu.sync_copy(src_hbm.at[pl.ds(off, C)], srcs[b])

            pltpu.make_async_copy(rows[b], acc_sh.at[dsts[b]], ssem).wait()
            if with_count:
                pltpu.make_async_copy(ones_v, cnt_sh.at[dsts[b]],
                                      csem).wait()

            @pl.when(i + NBUF < NCHUNK)
            def _():
                off = base + (i + NBUF) * C
                pltpu.sync_copy(dst_hbm.at[pl.ds(off, C)], dsts[b])
                pltpu.async_copy(feat_hbm.at[srcs[b]], rows[b], gsem[b])
        return carry
    lax.fori_loop(0, NCHUNK // NBUF, quad, 0)

    plsc.subcore_barrier()

    pltpu.sync_copy(acc_sh.at[pl.ds(row0, RPS)], acc_out.at[c, pl.ds(row0, RPS)])
    if with_count:
        pltpu.sync_copy(cnt_sh.at[pl.ds(crow0, CRPS)],
                        cnt_out.at[c, pl.ds(crow0, CRPS)])


def _make_sc_agg(with_count):
    mesh = plsc.VectorSubcoreMesh(core_axis_name="c", subcore_axis_name="s")
    out_type = [jax.ShapeDtypeStruct((NC, NP, H), jnp.float32)]
    scratch = []
    if with_count:
        out_type.append(jax.ShapeDtypeStruct((NC, CNP), jnp.float32))
        scratch += [
            pltpu.VMEM((C,), jnp.float32),
            pltpu.VMEM((CRPS,), jnp.float32),
        ]
    scratch += [
        pltpu.VMEM((ZB, H), jnp.float32),
        pltpu.VMEM_SHARED((NP, H), jnp.float32),
    ]
    if with_count:
        scratch.append(pltpu.VMEM_SHARED((CNP,), jnp.float32))
    scratch += [pltpu.VMEM((C,), jnp.int32)] * NBUF
    scratch += [pltpu.VMEM((C,), jnp.int32)] * NBUF
    scratch += [pltpu.VMEM((C, H), jnp.float32)] * NBUF
    scratch += [pltpu.SemaphoreType.DMA] * NBUF
    scratch.append(pltpu.SemaphoreType.DMA)
    if with_count:
        scratch.append(pltpu.SemaphoreType.DMA)
    return pl.kernel(
        functools.partial(_sc_agg_body, with_count),
        out_type=out_type, mesh=mesh, scratch_types=scratch,
    )


_sc_agg_l1 = _make_sc_agg(True)
_sc_agg_l2 = _make_sc_agg(False)


BLK = 632


def _tc_layer_body(acc_ref, cnt_ref, feat_ref, wl_ref, b_ref, wr_ref, out_ref):
    acc = acc_ref[0] + acc_ref[1]
    cnt = cnt_ref[0] + cnt_ref[1]
    mean = acc / jnp.maximum(cnt, 1.0)
    h = lax.dot_general(mean, wl_ref[...], (((1,), (1,)), ((), ())),
                        preferred_element_type=jnp.float32)
    h = h + lax.dot_general(feat_ref[...], wr_ref[...], (((1,), (1,)), ((), ())),
                            preferred_element_type=jnp.float32)
    h = h + b_ref[...]
    out_ref[...] = jnp.maximum(h, 0.0)


def _tc_layer(acc, cnt, feat, W_l, b_l, W_r):
    return pl.pallas_call(
        _tc_layer_body,
        grid=(NP // BLK,),
        in_specs=[
            pl.BlockSpec((NC, BLK, H), lambda i: (0, i, 0)),
            pl.BlockSpec((NC, BLK, 1), lambda i: (0, i, 0)),
            pl.BlockSpec((BLK, H), lambda i: (i, 0)),
            pl.BlockSpec((H, H), lambda i: (0, 0)),
            pl.BlockSpec((1, H), lambda i: (0, 0)),
            pl.BlockSpec((H, H), lambda i: (0, 0)),
        ],
        out_specs=pl.BlockSpec((BLK, H), lambda i: (i, 0)),
        out_shape=jax.ShapeDtypeStruct((NP, H), jnp.float32),
    )(acc, cnt[:, :NP].reshape(NC, NP, 1), feat, W_l, b_l.reshape(1, H), W_r)


def kernel(x, edge_index, W_l1, b_l1, W_r1, W_l2, b_l2, W_r2):
    ei = edge_index.astype(jnp.int32)
    npad = EP - E
    pad_src = jnp.arange(npad, dtype=jnp.int32) * 37 % N
    pad_dst = N + (jnp.arange(npad, dtype=jnp.int32) % (NP - N))
    eip = jnp.concatenate([ei, jnp.stack([pad_src, pad_dst])], axis=1)
    src = eip[0]
    dst = eip[1]
    xp = jnp.pad(x, ((0, NP - N), (0, 0)))
    acc1, cnt = _sc_agg_l1(xp, src, dst)
    h1 = _tc_layer(acc1, cnt, xp, W_l1, b_l1, W_r1)
    (acc2,) = _sc_agg_l2(h1, src, dst)
    h2 = _tc_layer(acc2, cnt, h1, W_l2, b_l2, W_r2)
    return h2[:N]


# speedup vs baseline: 1.2783x; 1.2783x over previous
import functools

import jax
import jax.numpy as jnp
from jax import lax
from jax.experimental import pallas as pl
from jax.experimental.pallas import tpu as pltpu
from jax.experimental.pallas import tpu_sc as plsc

N = 10000
NP = 10112
CNP = 10240
H = 128
E = 320000

NC = 2
NS = 16
NW = NC * NS
EP = 327680
EPT = EP // NW
C = 128
NCHUNK = EPT // C
RPS = NP // NS
CRPS = CNP // NS
ZB = 79


def _sc_agg_body(with_count, feat_hbm, src_hbm, dst_hbm, *refs):
    if with_count:
        (acc_out, cnt_out, src_v, dst_v, src_b, dst_b, rows_v, rows_b,
         ones_v, zc_v, zb_v, acc_sh, cnt_sh, sem, semb, ssem, csem) = refs
    else:
        (acc_out, src_v, dst_v, src_b, dst_b, rows_v, rows_b, zb_v,
         acc_sh, sem, semb, ssem) = refs

    c = lax.axis_index("c")
    s = lax.axis_index("s")
    wid = c * NS + s

    zeros16 = jnp.zeros((16,), jnp.float32)

    def zrow(r, carry):
        for cc in range(H // 16):
            zb_v[r, pl.ds(cc * 16, 16)] = zeros16
        return carry
    lax.fori_loop(0, ZB, zrow, 0)

    if with_count:
        ones16 = jnp.ones((16,), jnp.float32)

        def orow(r, carry):
            ones_v[pl.ds(r * 16, 16)] = ones16
            return carry
        lax.fori_loop(0, C // 16, orow, 0)

        def zcrow(r, carry):
            zc_v[pl.ds(r * 16, 16)] = zeros16
            return carry
        lax.fori_loop(0, CRPS // 16, zcrow, 0)

    row0 = s * RPS
    for z in range(RPS // ZB):
        pltpu.sync_copy(zb_v, acc_sh.at[pl.ds(row0 + z * ZB, ZB)])
    crow0 = s * CRPS
    if with_count:
        pltpu.sync_copy(zc_v, cnt_sh.at[pl.ds(crow0, CRPS)])
    plsc.subcore_barrier()

    base = wid * EPT
    pltpu.sync_copy(src_hbm.at[pl.ds(base, C)], src_v)
    pltpu.sync_copy(dst_hbm.at[pl.ds(base, C)], dst_v)
    pltpu.async_copy(feat_hbm.at[src_v], rows_v, sem)
    pltpu.sync_copy(src_hbm.at[pl.ds(base + C, C)], src_b)
    pltpu.sync_copy(dst_hbm.at[pl.ds(base + C, C)], dst_b)
    pltpu.async_copy(feat_hbm.at[src_b], rows_b, semb)

    def pair(g, carry):
        i = 2 * g

        pltpu.make_async_copy(feat_hbm.at[src_v], rows_v, sem).wait()
        if with_count:
            pltpu.async_copy(ones_v, cnt_sh.at[dst_v], csem, add=True)
        pltpu.async_copy(rows_v, acc_sh.at[dst_v], ssem, add=True)

        @pl.when(i + 2 < NCHUNK)
        def _():
            offa = base + (i + 2) * C
            pltpu.sync_copy(src_hbm.at[pl.ds(offa, C)], src_v)

        pltpu.make_async_copy(rows_v, acc_sh.at[dst_v], ssem).wait()
        if with_count:
            pltpu.make_async_copy(ones_v, cnt_sh.at[dst_v], csem).wait()

        @pl.when(i + 2 < NCHUNK)
        def _():
            offa = base + (i + 2) * C
            pltpu.sync_copy(dst_hbm.at[pl.ds(offa, C)], dst_v)
            pltpu.async_copy(feat_hbm.at[src_v], rows_v, sem)

        pltpu.make_async_copy(feat_hbm.at[src_b], rows_b, semb).wait()
        if with_count:
            pltpu.async_copy(ones_v, cnt_sh.at[dst_b], csem, add=True)
        pltpu.async_copy(rows_b, acc_sh.at[dst_b], ssem, add=True)

        @pl.when(i + 3 < NCHUNK)
        def _():
            offb = base + (i + 3) * C
            pltpu.sync_copy(src_hbm.at[pl.ds(offb, C)], src_b)

        pltpu.make_async_copy(rows_b, acc_sh.at[dst_b], ssem).wait()
        if with_count:
            pltpu.make_async_copy(ones_v, cnt_sh.at[dst_b], csem).wait()

        @pl.when(i + 3 < NCHUNK)
        def _():
            offb = base + (i + 3) * C
            pltpu.sync_copy(dst_hbm.at[pl.ds(offb, C)], dst_b)
            pltpu.async_copy(feat_hbm.at[src_b], rows_b, semb)
        return carry
    lax.fori_loop(0, NCHUNK // 2, pair, 0)

    plsc.subcore_barrier()

    pltpu.sync_copy(acc_sh.at[pl.ds(row0, RPS)], acc_out.at[c, pl.ds(row0, RPS)])
    if with_count:
        pltpu.sync_copy(cnt_sh.at[pl.ds(crow0, CRPS)],
                        cnt_out.at[c, pl.ds(crow0, CRPS)])


def _make_sc_agg(with_count):
    mesh = plsc.VectorSubcoreMesh(core_axis_name="c", subcore_axis_name="s")
    out_type = [jax.ShapeDtypeStruct((NC, NP, H), jnp.float32)]
    scratch = [
        pltpu.VMEM((C,), jnp.int32),
        pltpu.VMEM((C,), jnp.int32),
        pltpu.VMEM((C,), jnp.int32),
        pltpu.VMEM((C,), jnp.int32),
        pltpu.VMEM((C, H), jnp.float32),
        pltpu.VMEM((C, H), jnp.float32),
    ]
    if with_count:
        out_type.append(jax.ShapeDtypeStruct((NC, CNP), jnp.float32))
        scratch += [
            pltpu.VMEM((C,), jnp.float32),
            pltpu.VMEM((CRPS,), jnp.float32),
        ]
    scratch += [
        pltpu.VMEM((ZB, H), jnp.float32),
        pltpu.VMEM_SHARED((NP, H), jnp.float32),
    ]
    if with_count:
        scratch.append(pltpu.VMEM_SHARED((CNP,), jnp.float32))
    scratch.append(pltpu.SemaphoreType.DMA)
    scratch.append(pltpu.SemaphoreType.DMA)
    scratch.append(pltpu.SemaphoreType.DMA)
    if with_count:
        scratch.append(pltpu.SemaphoreType.DMA)
    return pl.kernel(
        functools.partial(_sc_agg_body, with_count),
        out_type=out_type, mesh=mesh, scratch_types=scratch,
    )


_sc_agg_l1 = _make_sc_agg(True)
_sc_agg_l2 = _make_sc_agg(False)


BLK = 2528


def _tc_layer_body(acc_ref, cnt_ref, feat_ref, wl_ref, b_ref, wr_ref, out_ref):
    acc = acc_ref[0] + acc_ref[1]
    cnt = cnt_ref[0] + cnt_ref[1]
    mean = acc / jnp.maximum(cnt, 1.0)
    h = lax.dot_general(mean, wl_ref[...], (((1,), (1,)), ((), ())),
                        preferred_element_type=jnp.float32)
    h = h + lax.dot_general(feat_ref[...], wr_ref[...], (((1,), (1,)), ((), ())),
                            preferred_element_type=jnp.float32)
    h = h + b_ref[...]
    out_ref[...] = jnp.maximum(h, 0.0)


def _tc_layer(acc, cnt, feat, W_l, b_l, W_r):
    return pl.pallas_call(
        _tc_layer_body,
        grid=(NP // BLK,),
        in_specs=[
            pl.BlockSpec((NC, BLK, H), lambda i: (0, i, 0)),
            pl.BlockSpec((NC, BLK, 1), lambda i: (0, i, 0)),
            pl.BlockSpec((BLK, H), lambda i: (i, 0)),
            pl.BlockSpec((H, H), lambda i: (0, 0)),
            pl.BlockSpec((1, H), lambda i: (0, 0)),
            pl.BlockSpec((H, H), lambda i: (0, 0)),
        ],
        out_specs=pl.BlockSpec((BLK, H), lambda i: (i, 0)),
        out_shape=jax.ShapeDtypeStruct((NP, H), jnp.float32),
    )(acc, cnt[:, :NP].reshape(NC, NP, 1), feat, W_l, b_l.reshape(1, H), W_r)


def kernel(x, edge_index, W_l1, b_l1, W_r1, W_l2, b_l2, W_r2):
    ei = edge_index.astype(jnp.int32)
    npad = EP - E
    pad_src = jnp.arange(npad, dtype=jnp.int32) * 37 % N
    pad_dst = N + (jnp.arange(npad, dtype=jnp.int32) % (NP - N))
    eip = jnp.concatenate([ei, jnp.stack([pad_src, pad_dst])], axis=1)
    src = eip[0]
    dst = eip[1]
    xp = jnp.pad(x, ((0, NP - N), (0, 0)))
    acc1, cnt = _sc_agg_l1(xp, src, dst)
    h1 = _tc_layer(acc1, cnt, xp, W_l1, b_l1, W_r1)
    (acc2,) = _sc_agg_l2(h1, src, dst)
    h2 = _tc_layer(acc2, cnt, h1, W_l2, b_l2, W_r2)
    return h2[:N]


# speedup vs baseline: 1.2800x; 1.0013x over previous
import functools

import jax
import jax.numpy as jnp
from jax import lax
from jax.experimental import pallas as pl
from jax.experimental.pallas import tpu as pltpu
from jax.experimental.pallas import tpu_sc as plsc

N = 10000
NP = 10112
CNP = 10240
H = 128
E = 320000

NC = 2
NS = 16
NW = NC * NS
EP = 327680
EPT = EP // NW
C = 128
NCHUNK = EPT // C
RPS = NP // NS
CRPS = CNP // NS
ZB = 79


def _sc_agg_body(with_count, feat_hbm, src_hbm, dst_hbm, *refs):
    if with_count:
        (acc_out, cnt_out, src_v, dst_v, src_b, dst_b, rows_v, rows_b,
         ones_v, zc_v, zb_v, acc_sh, cnt_sh, sem, semb, ssem, csem) = refs
    else:
        (acc_out, src_v, dst_v, src_b, dst_b, rows_v, rows_b, zb_v,
         acc_sh, sem, semb, ssem) = refs

    c = lax.axis_index("c")
    s = lax.axis_index("s")
    wid = c * NS + s

    zeros16 = jnp.zeros((16,), jnp.float32)

    def zrow(r, carry):
        for cc in range(H // 16):
            zb_v[r, pl.ds(cc * 16, 16)] = zeros16
        return carry
    lax.fori_loop(0, ZB, zrow, 0)

    if with_count:
        ones16 = jnp.ones((16,), jnp.float32)

        def orow(r, carry):
            ones_v[pl.ds(r * 16, 16)] = ones16
            return carry
        lax.fori_loop(0, C // 16, orow, 0)

        def zcrow(r, carry):
            zc_v[pl.ds(r * 16, 16)] = zeros16
            return carry
        lax.fori_loop(0, CRPS // 16, zcrow, 0)

    row0 = s * RPS
    for z in range(RPS // ZB):
        pltpu.sync_copy(zb_v, acc_sh.at[pl.ds(row0 + z * ZB, ZB)])
    crow0 = s * CRPS
    if with_count:
        pltpu.sync_copy(zc_v, cnt_sh.at[pl.ds(crow0, CRPS)])
    plsc.subcore_barrier()

    base = wid * EPT
    pltpu.sync_copy(src_hbm.at[pl.ds(base, C)], src_v)
    pltpu.sync_copy(dst_hbm.at[pl.ds(base, C)], dst_v)
    pltpu.async_copy(feat_hbm.at[src_v], rows_v, sem)
    pltpu.sync_copy(src_hbm.at[pl.ds(base + C, C)], src_b)
    pltpu.sync_copy(dst_hbm.at[pl.ds(base + C, C)], dst_b)
    pltpu.async_copy(feat_hbm.at[src_b], rows_b, semb)

    def pair(g, carry):
        i = 2 * g

        pltpu.make_async_copy(feat_hbm.at[src_v], rows_v, sem).wait()
        if with_count:
            pltpu.async_copy(ones_v, cnt_sh.at[dst_v], csem, add=True)
        pltpu.async_copy(rows_v, acc_sh.at[dst_v], ssem, add=True)

        @pl.when(i + 2 < NCHUNK)
        def _():
            offa = base + (i + 2) * C
            pltpu.sync_copy(src_hbm.at[pl.ds(offa, C)], src_v)

        pltpu.make_async_copy(rows_v, acc_sh.at[dst_v], ssem).wait()
        if with_count:
            pltpu.make_async_copy(ones_v, cnt_sh.at[dst_v], csem).wait()

        @pl.when(i + 2 < NCHUNK)
        def _():
            offa = base + (i + 2) * C
            pltpu.sync_copy(dst_hbm.at[pl.ds(offa, C)], dst_v)
            pltpu.async_copy(feat_hbm.at[src_v], rows_v, sem)

        pltpu.make_async_copy(feat_hbm.at[src_b], rows_b, semb).wait()
        if with_count:
            pltpu.async_copy(ones_v, cnt_sh.at[dst_b], csem, add=True)
        pltpu.async_copy(rows_b, acc_sh.at[dst_b], ssem, add=True)

        @pl.when(i + 3 < NCHUNK)
        def _():
            offb = base + (i + 3) * C
            pltpu.sync_copy(src_hbm.at[pl.ds(offb, C)], src_b)

        pltpu.make_async_copy(rows_b, acc_sh.at[dst_b], ssem).wait()
        if with_count:
            pltpu.make_async_copy(ones_v, cnt_sh.at[dst_b], csem).wait()

        @pl.when(i + 3 < NCHUNK)
        def _():
            offb = base + (i + 3) * C
            pltpu.sync_copy(dst_hbm.at[pl.ds(offb, C)], dst_b)
            pltpu.async_copy(feat_hbm.at[src_b], rows_b, semb)
        return carry
    lax.fori_loop(0, NCHUNK // 2, pair, 0)

    plsc.subcore_barrier()

    pltpu.sync_copy(acc_sh.at[pl.ds(row0, RPS)], acc_out.at[c, pl.ds(row0, RPS)])
    if with_count:
        pltpu.sync_copy(cnt_sh.at[pl.ds(crow0, CRPS)],
                        cnt_out.at[c, pl.ds(crow0, CRPS)])


def _make_sc_agg(with_count):
    mesh = plsc.VectorSubcoreMesh(core_axis_name="c", subcore_axis_name="s")
    out_type = [jax.ShapeDtypeStruct((NC, NP, H), jnp.float32)]
    scratch = [
        pltpu.VMEM((C,), jnp.int32),
        pltpu.VMEM((C,), jnp.int32),
        pltpu.VMEM((C,), jnp.int32),
        pltpu.VMEM((C,), jnp.int32),
        pltpu.VMEM((C, H), jnp.float32),
        pltpu.VMEM((C, H), jnp.float32),
    ]
    if with_count:
        out_type.append(jax.ShapeDtypeStruct((NC, CNP), jnp.float32))
        scratch += [
            pltpu.VMEM((C,), jnp.float32),
            pltpu.VMEM((CRPS,), jnp.float32),
        ]
    scratch += [
        pltpu.VMEM((ZB, H), jnp.float32),
        pltpu.VMEM_SHARED((NP, H), jnp.float32),
    ]
    if with_count:
        scratch.append(pltpu.VMEM_SHARED((CNP,), jnp.float32))
    scratch.append(pltpu.SemaphoreType.DMA)
    scratch.append(pltpu.SemaphoreType.DMA)
    scratch.append(pltpu.SemaphoreType.DMA)
    if with_count:
        scratch.append(pltpu.SemaphoreType.DMA)
    return pl.kernel(
        functools.partial(_sc_agg_body, with_count),
        out_type=out_type, mesh=mesh, scratch_types=scratch,
    )


_sc_agg_l1 = _make_sc_agg(True)
_sc_agg_l2 = _make_sc_agg(False)


BLK = 5056


def _tc_layer_body(acc_ref, cnt_ref, feat_ref, wl_ref, b_ref, wr_ref, out_ref):
    acc = acc_ref[0] + acc_ref[1]
    cnt = cnt_ref[0] + cnt_ref[1]
    mean = acc / jnp.maximum(cnt, 1.0)
    h = lax.dot_general(mean, wl_ref[...], (((1,), (1,)), ((), ())),
                        preferred_element_type=jnp.float32)
    h = h + lax.dot_general(feat_ref[...], wr_ref[...], (((1,), (1,)), ((), ())),
                            preferred_element_type=jnp.float32)
    h = h + b_ref[...]
    out_ref[...] = jnp.maximum(h, 0.0)


def _tc_layer(acc, cnt, feat, W_l, b_l, W_r):
    return pl.pallas_call(
        _tc_layer_body,
        grid=(NP // BLK,),
        in_specs=[
            pl.BlockSpec((NC, BLK, H), lambda i: (0, i, 0)),
            pl.BlockSpec((NC, BLK, 1), lambda i: (0, i, 0)),
            pl.BlockSpec((BLK, H), lambda i: (i, 0)),
            pl.BlockSpec((H, H), lambda i: (0, 0)),
            pl.BlockSpec((1, H), lambda i: (0, 0)),
            pl.BlockSpec((H, H), lambda i: (0, 0)),
        ],
        out_specs=pl.BlockSpec((BLK, H), lambda i: (i, 0)),
        out_shape=jax.ShapeDtypeStruct((NP, H), jnp.float32),
    )(acc, cnt[:, :NP].reshape(NC, NP, 1), feat, W_l, b_l.reshape(1, H), W_r)


def kernel(x, edge_index, W_l1, b_l1, W_r1, W_l2, b_l2, W_r2):
    ei = edge_index.astype(jnp.int32)
    npad = EP - E
    pad_src = jnp.arange(npad, dtype=jnp.int32) * 37 % N
    pad_dst = N + (jnp.arange(npad, dtype=jnp.int32) % (NP - N))
    eip = jnp.concatenate([ei, jnp.stack([pad_src, pad_dst])], axis=1)
    src = eip[0]
    dst = eip[1]
    xp = jnp.pad(x, ((0, NP - N), (0, 0)))
    acc1, cnt = _sc_agg_l1(xp, src, dst)
    h1 = _tc_layer(acc1, cnt, xp, W_l1, b_l1, W_r1)
    (acc2,) = _sc_agg_l2(h1, src, dst)
    h2 = _tc_layer(acc2, cnt, h1, W_l2, b_l2, W_r2)
    return h2[:N]
